# Initial kernel scaffold; baseline (speedup 1.0000x reference)
import functools

import jax
import jax.numpy as jnp
from jax import lax
from jax.experimental import pallas as pl
from jax.experimental.pallas import tpu as pltpu
from jax.experimental.pallas import tpu_sc as plsc

N = 10000
E = 320000
D = 128
H = 4
C = 128
NUM_CLASSES = 64
NUM_GRAPHS = 16

NC = 2
NS = 16
NW = NC * NS
CHUNK = 128
NCHUNK = 79
EPW = CHUNK * NCHUNK
EPAD = NW * EPW - E
NP = 10240
RPT = NP // NS
AW = 144

_f32 = jnp.float32
_mesh = plsc.VectorSubcoreMesh(
    core_axis_name="c", subcore_axis_name="s", num_cores=NC, num_subcores=NS)


def _worker_id():
  return lax.axis_index("s") * NC + lax.axis_index("c")


@functools.partial(
    pl.kernel,
    out_type=jax.ShapeDtypeStruct((NC, NP, 16), _f32),
    mesh=_mesh,
    scratch_types=[
        pltpu.VMEM((CHUNK,), jnp.int32),
        pltpu.VMEM((CHUNK, 16), _f32),
        pltpu.VMEM_SHARED((NP, 16), _f32),
    ],
)
def _sc_deg(dst_hbm, ones_hbm, z16_hbm, out_hbm, didx, ones_v, acc):
  c = lax.axis_index("c")
  s = lax.axis_index("s")
  wid = _worker_id()
  pltpu.sync_copy(ones_hbm, ones_v)

  @pl.loop(0, 5)
  def _zero(i):
    pltpu.sync_copy(z16_hbm, acc.at[pl.ds(s * RPT + i * CHUNK, CHUNK)])

  plsc.subcore_barrier()

  @pl.loop(0, NCHUNK)
  def _edges(k):
    pltpu.sync_copy(dst_hbm.at[pl.ds(wid * EPW + k * CHUNK, CHUNK)], didx)
    pltpu.sync_copy(ones_v, acc.at[didx], add=True)

  plsc.subcore_barrier()
  pltpu.sync_copy(acc.at[pl.ds(s * RPT, RPT)], out_hbm.at[c, pl.ds(s * RPT, RPT)])


@functools.partial(
    pl.kernel,
    out_type=jax.ShapeDtypeStruct((NC, NP, C), _f32),
    mesh=_mesh,
    scratch_types=[
        pltpu.VMEM((CHUNK,), jnp.int32),
        pltpu.VMEM((CHUNK,), jnp.int32),
        pltpu.VMEM((CHUNK, C), _f32),
        pltpu.VMEM_SHARED((NP, C), _f32),
    ],
)
def _sc_seg(y_hbm, src_hbm, dst_hbm, z128_hbm, out_hbm, sidx, didx, rows, acc):
  c = lax.axis_index("c")
  s = lax.axis_index("s")
  wid = _worker_id()

  @pl.loop(0, 5)
  def _zero(i):
    pltpu.sync_copy(z128_hbm, acc.at[pl.ds(s * RPT + i * CHUNK, CHUNK)])

  plsc.subcore_barrier()

  @pl.loop(0, NCHUNK)
  def _edges(k):
    base = wid * EPW + k * CHUNK
    pltpu.sync_copy(src_hbm.at[pl.ds(base, CHUNK)], sidx)
    pltpu.sync_copy(dst_hbm.at[pl.ds(base, CHUNK)], didx)
    pltpu.sync_copy(y_hbm.at[sidx], rows)
    pltpu.sync_copy(rows, acc.at[didx], add=True)

  plsc.subcore_barrier()
  pltpu.sync_copy(acc.at[pl.ds(s * RPT, RPT)], out_hbm.at[c, pl.ds(s * RPT, RPT)])


def _make_sc_att(h):
  @functools.partial(
      pl.kernel,
      out_type=jax.ShapeDtypeStruct((NC, NP, AW), _f32),
      mesh=_mesh,
      scratch_types=[
          pltpu.VMEM((CHUNK,), jnp.int32),
          pltpu.VMEM((CHUNK,), jnp.int32),
          pltpu.VMEM((CHUNK, C), _f32),
          pltpu.VMEM((CHUNK, AW), _f32),
          pltpu.VMEM((N,), _f32),
          pltpu.VMEM((N,), _f32),
          pltpu.VMEM((16,), _f32),
          pltpu.VMEM((16,), _f32),
          pltpu.VMEM_SHARED((NP, AW), _f32),
      ],
  )
  def _sc_att(xlh_hbm, asrc_hbm, adst_hbm, k16_hbm, src_hbm, dst_hbm, z144_hbm,
              out_hbm, sidx, didx, rows, srow, at_s, at_d, kv, wbuf, acc):
    c = lax.axis_index("c")
    s = lax.axis_index("s")
    wid = _worker_id()
    pltpu.sync_copy(asrc_hbm, at_s)
    pltpu.sync_copy(adst_hbm, at_d)
    pltpu.sync_copy(k16_hbm, kv)

    @pl.loop(0, 5)
    def _zero(i):
      pltpu.sync_copy(z144_hbm, acc.at[pl.ds(s * RPT + i * CHUNK, CHUNK)])

    plsc.subcore_barrier()
    ohv = jnp.where(lax.iota(jnp.int32, 16) == 0, 1.0, 0.0).astype(_f32)

    @pl.loop(0, NCHUNK)
    def _edges(k):
      base = wid * EPW + k * CHUNK
      pltpu.sync_copy(src_hbm.at[pl.ds(base, CHUNK)], sidx)
      pltpu.sync_copy(dst_hbm.at[pl.ds(base, CHUNK)], didx)
      pltpu.sync_copy(xlh_hbm.at[sidx], rows)

      @pl.loop(0, CHUNK // 16)
      def _group(g):
        s16 = sidx[pl.ds(g * 16, 16)]
        d16 = didx[pl.ds(g * 16, 16)]
        e = plsc.load_gather(at_s, [s16]) + plsc.load_gather(at_d, [d16])
        e = jnp.where(e >= 0, e, 0.2 * e)
        wbuf[...] = jnp.exp(e - kv[h])

        @pl.loop(0, 16)
        def _edge(j):
          wj = wbuf[j]
          eidx = g * 16 + j
          for r in range(C // 16):
            srow[eidx, pl.ds(r * 16, 16)] = wj * rows[eidx, pl.ds(r * 16, 16)]
          srow[eidx, pl.ds(C, 16)] = wj * ohv

      pltpu.sync_copy(srow, acc.at[didx], add=True)

    plsc.subcore_barrier()
    pltpu.sync_copy(acc.at[pl.ds(s * RPT, RPT)],
                    out_hbm.at[c, pl.ds(s * RPT, RPT)])

  return _sc_att


_sc_att_heads = [_make_sc_att(h) for h in range(H)]

_HI = jax.lax.Precision.HIGHEST


def _dot(a, b):
  return jax.lax.dot_general(a, b, (((1,), (0,)), ((), ())), precision=_HI,
                             preferred_element_type=_f32)


def _tc_a_body(x_ref, w1_ref, dp_ref, y1_ref, dinv_ref):
  deg = dp_ref[0, :N, 0:1] + dp_ref[1, :N, 0:1] + 1.0
  dinv = jax.lax.rsqrt(deg)
  dinv_ref[...] = dinv
  y1_ref[...] = dinv * _dot(x_ref[...], w1_ref[...])


def _tc_a(x, w1, degpart):
  return pl.pallas_call(
      _tc_a_body,
      out_shape=[
          jax.ShapeDtypeStruct((N, C), _f32),
          jax.ShapeDtypeStruct((N, 1), _f32),
      ],
  )(x, w1, degpart)


def _tc_b_body(sp_ref, y1_ref, dinv_ref, b1_ref, w2_ref, asv_ref, adv_ref,
               xlh_ref, asrc_ref, adst_ref, k_ref, wself_ref):
  s1 = sp_ref[0, :N, :] + sp_ref[1, :N, :] + y1_ref[...]
  x1 = jnp.maximum(dinv_ref[...] * s1 + b1_ref[...], 0.0)
  kacc = jnp.zeros((1, 16), _f32)
  lane = lax.broadcasted_iota(jnp.int32, (1, 16), 1)
  for h in range(H):
    xl_h = _dot(x1, w2_ref[:, h * C:(h + 1) * C])
    xlh_ref[h, :, :] = xl_h
    a_s = _dot(xl_h, jnp.reshape(asv_ref[h, :], (C, 1)))
    a_d = _dot(xl_h, jnp.reshape(adv_ref[h, :], (C, 1)))
    asrc_ref[:, h:h + 1] = a_s
    adst_ref[:, h:h + 1] = a_d
    kh = jnp.maximum(jnp.max(a_s) + jnp.max(a_d), 0.0)
    kacc = kacc + jnp.where(lane == h, kh, 0.0)
    e_self = a_s + a_d
    e_self = jnp.where(e_self >= 0, e_self, 0.2 * e_self)
    wself_ref[:, h:h + 1] = jnp.exp(e_self - kh)
  k_ref[...] = kacc


def _tc_b(s1part, y1, dinv, b1, w2, att_src, att_dst):
  return pl.pallas_call(
      _tc_b_body,
      out_shape=[
          jax.ShapeDtypeStruct((H, N, C), _f32),
          jax.ShapeDtypeStruct((N, H), _f32),
          jax.ShapeDtypeStruct((N, H), _f32),
          jax.ShapeDtypeStruct((1, 16), _f32),
          jax.ShapeDtypeStruct((N, H), _f32),
      ],
  )(s1part, y1, dinv, b1, w2, att_src, att_dst)


_DBLK = 1000


def _tc_d_body(a0_ref, a1_ref, a2_ref, a3_ref, xlh_ref, ws_ref, dinv_ref,
               w3_ref, b2_ref, y3_ref):
  att = (a0_ref, a1_ref, a2_ref, a3_ref)
  acc = jnp.zeros((_DBLK, C), _f32)
  for h in range(H):
    a = att[h]
    num = (a[0, :, 0:C] + a[1, :, 0:C]
           + ws_ref[:, h:h + 1] * xlh_ref[h, :, :])
    den = a[0, :, C:C + 1] + a[1, :, C:C + 1] + ws_ref[:, h:h + 1] + 1e-16
    acc = acc + num / den
  x2 = jnp.maximum(acc * (1.0 / H) + b2_ref[...], 0.0)
  y3_ref[...] = dinv_ref[...] * _dot(x2, w3_ref[...])


def _tc_d(attps, xlh, wself, dinv, w3, b2):
  att_spec = pl.BlockSpec((2, _DBLK, AW), lambda i: (0, i, 0))
  return pl.pallas_call(
      _tc_d_body,
      grid=(N // _DBLK,),
      in_specs=[att_spec, att_spec, att_spec, att_spec,
                pl.BlockSpec((H, _DBLK, C), lambda i: (0, i, 0)),
                pl.BlockSpec((_DBLK, H), lambda i: (i, 0)),
                pl.BlockSpec((_DBLK, 1), lambda i: (i, 0)),
                pl.BlockSpec((C, C), lambda i: (0, 0)),
                pl.BlockSpec((1, C), lambda i: (0, 0))],
      out_specs=pl.BlockSpec((_DBLK, C), lambda i: (i, 0)),
      out_shape=jax.ShapeDtypeStruct((N, C), _f32),
  )(*attps, xlh, wself, dinv, w3, b2)


def _tc_e_body(sp_ref, y3_ref, dinv_ref, b3_ref, x_ref, wsk_ref, bsk_ref,
               batch_ref, wf_ref, bf_ref, out_ref):
  s3 = sp_ref[0, :N, :] + sp_ref[1, :N, :] + y3_ref[...]
  x3 = (dinv_ref[...] * s3 + b3_ref[...]
        + _dot(x_ref[...], wsk_ref[...]) + bsk_ref[...])
  gids = lax.broadcasted_iota(jnp.int32, (NUM_GRAPHS, N), 0)
  mask = (batch_ref[...] == gids).astype(_f32)
  sums = _dot(mask, x3)
  cnts = jnp.sum(mask, axis=1, keepdims=True)
  pooled = sums / jnp.maximum(cnts, 1.0)
  out_ref[...] = _dot(pooled, wf_ref[...]) + bf_ref[...]


def _tc_e(s3part, y3, dinv, b3, x, wsk, bsk, batch, wf, bf):
  return pl.pallas_call(
      _tc_e_body,
      out_shape=jax.ShapeDtypeStruct((NUM_GRAPHS, NUM_CLASSES), _f32),
  )(s3part, y3, dinv, b3, x, wsk, bsk, batch, wf, bf)


def kernel(x, edge_index, batch, W1, b1, W2, att_src, att_dst, b2, W3, b3,
           Wsk, bsk, Wf, bf):
  i32 = jnp.int32
  srcp = jnp.concatenate([edge_index[0], jnp.zeros((EPAD,), i32)])
  dstp = jnp.concatenate([edge_index[1], jnp.full((EPAD,), N, i32)])
  ones16 = jnp.ones((CHUNK, 16), _f32)
  z16 = jnp.zeros((CHUNK, 16), _f32)
  z128 = jnp.zeros((CHUNK, C), _f32)
  z144 = jnp.zeros((CHUNK, AW), _f32)

  degpart = _sc_deg(dstp, ones16, z16)
  y1, dinv = _tc_a(x, W1, degpart)
  s1part = _sc_seg(y1, srcp, dstp, z128)
  xlh, asrc, adst, k16, wself = _tc_b(
      s1part, y1, dinv, b1.reshape(1, C), W2, att_src, att_dst)
  k16f = k16.reshape(16)
  attps = [
      _sc_att_heads[h](xlh[h], asrc[:, h].reshape(N), adst[:, h].reshape(N),
                       k16f, srcp, dstp, z144)
      for h in range(H)
  ]
  y3 = _tc_d(attps, xlh, wself, dinv, W3, b2.reshape(1, C))
  s3part = _sc_seg(y3, srcp, dstp, z128)
  return _tc_e(s3part, y3, dinv, b3.reshape(1, C), x, Wsk,
               bsk.reshape(1, C), batch.reshape(1, N), Wf, bf.reshape(1, NUM_CLASSES))


# speedup vs baseline: 14.5616x; 14.5616x over previous
import dataclasses
import functools

import jax
import jax.numpy as jnp
from jax import lax
from jax.experimental import pallas as pl
from jax.experimental.pallas import tpu as pltpu
from jax.experimental.pallas import tpu_sc as plsc

N = 10000
E = 320000
D = 128
H = 4
C = 128
NUM_CLASSES = 64
NUM_GRAPHS = 16

NC = 2
NS = 16
NW = NC * NS
CHUNK = 128
NCHUNK = 79
EPW = CHUNK * NCHUNK
EPAD = NW * EPW - E
NP = 10240
RPT = NP // NS

_f32 = jnp.float32
_mesh = plsc.VectorSubcoreMesh(
    core_axis_name="c", subcore_axis_name="s", num_cores=NC, num_subcores=NS)

_sc_params = pltpu.CompilerParams()
if "needs_layout_passes" in pltpu.CompilerParams.__dataclass_fields__:
  _sc_params = dataclasses.replace(_sc_params, needs_layout_passes=False)


def _worker_id():
  return lax.axis_index("s") * NC + lax.axis_index("c")


def _splat(vref, i):
  return plsc.load_gather(vref, [jnp.full((16,), i, jnp.int32)])


@functools.partial(
    pl.kernel,
    out_type=jax.ShapeDtypeStruct((NC, NP, C), _f32),
    mesh=_mesh,
    compiler_params=_sc_params,
    scratch_types=[
        pltpu.VMEM((CHUNK,), jnp.int32),
        pltpu.VMEM((CHUNK, C), _f32),
        pltpu.VMEM_SHARED((NP, C), _f32),
    ],
)
def _sc_deg(dst_hbm, ones_hbm, z128_hbm, out_hbm, didx, ones_v, acc):
  c = lax.axis_index("c")
  s = lax.axis_index("s")
  wid = _worker_id()
  pltpu.sync_copy(z128_hbm, ones_v)

  @pl.loop(0, 5)
  def _zero(i):
    pltpu.sync_copy(ones_v, acc.at[pl.ds(s * RPT + i * CHUNK, CHUNK)])

  pltpu.sync_copy(ones_hbm, ones_v)
  plsc.subcore_barrier()

  @pl.loop(0, NCHUNK)
  def _edges(k):
    pltpu.sync_copy(dst_hbm.at[pl.ds(wid * EPW + k * CHUNK, CHUNK)], didx)
    pltpu.sync_copy(ones_v, acc.at[didx], add=True)

  plsc.subcore_barrier()

  @pl.loop(0, 5)
  def _wb(i):
    pltpu.sync_copy(acc.at[pl.ds(s * RPT + i * CHUNK, CHUNK)], ones_v)
    pltpu.sync_copy(ones_v, out_hbm.at[c, pl.ds(s * RPT + i * CHUNK, CHUNK)])


@functools.partial(
    pl.kernel,
    out_type=jax.ShapeDtypeStruct((NC, NP, C), _f32),
    mesh=_mesh,
    compiler_params=_sc_params,
    scratch_types=[
        pltpu.VMEM((CHUNK,), jnp.int32),
        pltpu.VMEM((CHUNK,), jnp.int32),
        pltpu.VMEM((CHUNK, C), _f32),
        pltpu.VMEM_SHARED((NP, C), _f32),
    ],
)
def _sc_seg(y_hbm, src_hbm, dst_hbm, z128_hbm, out_hbm, sidx, didx, rows, acc):
  c = lax.axis_index("c")
  s = lax.axis_index("s")
  wid = _worker_id()
  pltpu.sync_copy(z128_hbm, rows)

  @pl.loop(0, 5)
  def _zero(i):
    pltpu.sync_copy(rows, acc.at[pl.ds(s * RPT + i * CHUNK, CHUNK)])

  plsc.subcore_barrier()

  @pl.loop(0, NCHUNK)
  def _edges(k):
    base = wid * EPW + k * CHUNK
    pltpu.sync_copy(src_hbm.at[pl.ds(base, CHUNK)], sidx)
    pltpu.sync_copy(dst_hbm.at[pl.ds(base, CHUNK)], didx)
    pltpu.sync_copy(y_hbm.at[sidx], rows)
    pltpu.sync_copy(rows, acc.at[didx], add=True)

  plsc.subcore_barrier()

  @pl.loop(0, 5)
  def _wb(i):
    pltpu.sync_copy(acc.at[pl.ds(s * RPT + i * CHUNK, CHUNK)], rows)
    pltpu.sync_copy(rows, out_hbm.at[c, pl.ds(s * RPT + i * CHUNK, CHUNK)])


NPK = NP // 8
PKT = NPK // NS
UPT = NP // NS * 16


@functools.partial(
    pl.kernel,
    out_type=[
        jax.ShapeDtypeStruct((NC, NS * UPT), _f32),
        jax.ShapeDtypeStruct((H, NW * EPW), _f32),
    ],
    mesh=_mesh,
    compiler_params=_sc_params,
    scratch_types=[
        pltpu.VMEM((CHUNK,), jnp.int32),
        pltpu.VMEM((CHUNK,), jnp.int32),
        pltpu.VMEM((CHUNK,), jnp.int32),
        pltpu.VMEM((CHUNK, C), _f32),
        pltpu.VMEM((H * CHUNK,), _f32),
        pltpu.VMEM((N * H,), _f32),
        pltpu.VMEM((N * H,), _f32),
        pltpu.VMEM((16,), _f32),
        pltpu.VMEM((PKT, C), _f32),
        pltpu.VMEM((UPT,), _f32),
        pltpu.VMEM_SHARED((NPK, C), _f32),
    ],
)
def _sc_attden(asrc_hbm, adst_hbm, k16_hbm, src_hbm, dst_hbm, z128_hbm,
               out_hbm, w_hbm, sidx, didx, didx8, wrow, wout, at_s, at_d, kv,
               pk, ub, acc):
  c = lax.axis_index("c")
  s = lax.axis_index("s")
  wid = _worker_id()
  pltpu.sync_copy(asrc_hbm, at_s)
  pltpu.sync_copy(adst_hbm, at_d)
  pltpu.sync_copy(k16_hbm, kv)
  pltpu.sync_copy(z128_hbm.at[pl.ds(0, PKT)], pk)
  pltpu.sync_copy(pk, acc.at[pl.ds(s * PKT, PKT)])

  @pl.loop(0, CHUNK)
  def _zrow(i):
    for r in range(C // 16):
      wrow[i, pl.ds(r * 16, 16)] = jnp.zeros((16,), _f32)

  plsc.subcore_barrier()
  lanes = lax.iota(jnp.int32, 16)
  zeros16 = jnp.zeros((16,), _f32)

  @pl.loop(0, NCHUNK)
  def _edges(k):
    base = wid * EPW + k * CHUNK
    pltpu.sync_copy(src_hbm.at[pl.ds(base, CHUNK)], sidx)
    pltpu.sync_copy(dst_hbm.at[pl.ds(base, CHUNK)], didx)

    @pl.loop(0, CHUNK // 16)
    def _group(g):
      s16 = sidx[pl.ds(g * 16, 16)]
      d16 = didx[pl.ds(g * 16, 16)]
      didx8[pl.ds(g * 16, 16)] = lax.shift_right_logical(d16, 3)
      rowids = g * 16 + lanes
      col = lax.shift_left(jnp.bitwise_and(d16, 7), 4)
      for h in range(H):
        e = (plsc.load_gather(at_s, [s16 * H + h])
             + plsc.load_gather(at_d, [d16 * H + h]))
        e = jnp.where(e >= 0, e, 0.2 * e)
        w = jnp.exp(e - _splat(kv, h))
        wout[pl.ds(h * CHUNK + g * 16, 16)] = w
        plsc.store_scatter(wrow, [rowids, col + h], w)

    pltpu.sync_copy(wrow, acc.at[didx8], add=True)
    for h in range(H):
      pltpu.sync_copy(wout.at[pl.ds(h * CHUNK, CHUNK)],
                      w_hbm.at[h, pl.ds(base, CHUNK)])

    @pl.loop(0, CHUNK // 16)
    def _rezero(g):
      d16 = didx[pl.ds(g * 16, 16)]
      rowids = g * 16 + lanes
      col = lax.shift_left(jnp.bitwise_and(d16, 7), 4)
      for h in range(H):
        plsc.store_scatter(wrow, [rowids, col + h], zeros16)

  plsc.subcore_barrier()
  pltpu.sync_copy(acc.at[pl.ds(s * PKT, PKT)], pk)

  @pl.loop(0, NP // NS)
  def _unpack(i):
    r = lax.shift_right_logical(i, 3)
    q = lax.shift_left(jnp.bitwise_and(i, 7), 4)
    ub[pl.ds(i * 16, 16)] = pk[r, pl.ds(q, 16)]

  pltpu.sync_copy(ub, out_hbm.at[c, pl.ds(s * UPT, UPT)])


@functools.partial(
    pl.kernel,
    out_type=jax.ShapeDtypeStruct((NC, NP, C), _f32),
    mesh=_mesh,
    compiler_params=_sc_params,
    scratch_types=[
        pltpu.VMEM((CHUNK,), jnp.int32),
        pltpu.VMEM((CHUNK,), jnp.int32),
        pltpu.VMEM((CHUNK, C), _f32),
        pltpu.VMEM((CHUNK,), _f32),
        pltpu.VMEM((N,), _f32),
        pltpu.VMEM((16,), _f32),
        pltpu.VMEM_SHARED((NP, C), _f32),
    ],
)
def _sc_attagg(xlh_hbm, wh_hbm, rden_hbm, src_hbm, dst_hbm, z128_hbm,
               out_hbm, sidx, didx, rows, wchunk, rden, vbuf, acc):
  c = lax.axis_index("c")
  s = lax.axis_index("s")
  wid = _worker_id()
  pltpu.sync_copy(rden_hbm, rden)
  pltpu.sync_copy(z128_hbm, rows)

  @pl.loop(0, 5)
  def _zero(i):
    pltpu.sync_copy(rows, acc.at[pl.ds(s * RPT + i * CHUNK, CHUNK)])

  plsc.subcore_barrier()

  @pl.loop(0, NCHUNK)
  def _edges(k):
    base = wid * EPW + k * CHUNK
    pltpu.sync_copy(src_hbm.at[pl.ds(base, CHUNK)], sidx)
    pltpu.sync_copy(dst_hbm.at[pl.ds(base, CHUNK)], didx)
    pltpu.sync_copy(wh_hbm.at[pl.ds(base, CHUNK)], wchunk)
    pltpu.sync_copy(xlh_hbm.at[sidx], rows)

    @pl.loop(0, CHUNK // 16)
    def _group(g):
      d16 = didx[pl.ds(g * 16, 16)]
      w16 = wchunk[pl.ds(g * 16, 16)]
      vbuf[...] = w16 * plsc.load_gather(rden, [d16])

      @pl.loop(0, 16)
      def _edge(j):
        vj = _splat(vbuf, j)
        eidx = g * 16 + j
        for r in range(C // 16):
          rows[eidx, pl.ds(r * 16, 16)] = vj * rows[eidx, pl.ds(r * 16, 16)]

    pltpu.sync_copy(rows, acc.at[didx], add=True)

  plsc.subcore_barrier()

  @pl.loop(0, 5)
  def _wb(i):
    pltpu.sync_copy(acc.at[pl.ds(s * RPT + i * CHUNK, CHUNK)], rows)
    pltpu.sync_copy(rows, out_hbm.at[c, pl.ds(s * RPT + i * CHUNK, CHUNK)])


_HI = jax.lax.Precision.HIGHEST


def _dot(a, b):
  return jax.lax.dot_general(a, b, (((1,), (0,)), ((), ())), precision=_HI,
                             preferred_element_type=_f32)


def _tc_a_body(x_ref, w1_ref, dp_ref, y1_ref, dinv_ref):
  deg = dp_ref[0, :N, 0:1] + dp_ref[1, :N, 0:1] + 1.0
  dinv = jax.lax.rsqrt(deg)
  dinv_ref[...] = dinv
  y1_ref[...] = dinv * _dot(x_ref[...], w1_ref[...])


def _tc_a(x, w1, degpart):
  return pl.pallas_call(
      _tc_a_body,
      out_shape=[
          jax.ShapeDtypeStruct((N, C), _f32),
          jax.ShapeDtypeStruct((N, 1), _f32),
      ],
  )(x, w1, degpart)


def _tc_b1_body(sp_ref, y1_ref, dinv_ref, b1_ref, x1_ref):
  s1 = sp_ref[0, :N, :] + sp_ref[1, :N, :] + y1_ref[...]
  x1_ref[...] = jnp.maximum(dinv_ref[...] * s1 + b1_ref[...], 0.0)


def _tc_b1(s1part, y1, dinv, b1):
  return pl.pallas_call(
      _tc_b1_body,
      out_shape=jax.ShapeDtypeStruct((N, C), _f32),
  )(s1part, y1, dinv, b1)


def _tc_b2_body(x1_ref, w2_ref, asv_ref, adv_ref,
                xlh_ref, asrc_ref, adst_ref, k_ref, wself_ref):
  h = pl.program_id(0)
  laneH = lax.broadcasted_iota(jnp.int32, (N, H), 1)
  lane16 = lax.broadcasted_iota(jnp.int32, (1, 16), 1)

  def acc_col(ref, val, lane):
    cur = jnp.where(lane == h, val, 0.0)

    @pl.when(h == 0)
    def _init():
      ref[...] = cur

    @pl.when(h != 0)
    def _add():
      ref[...] = ref[...] + cur

  xl_h = _dot(x1_ref[...], w2_ref[...])
  xlh_ref[0] = xl_h
  a_s = jnp.sum(xl_h * asv_ref[pl.ds(h, 1), :], axis=1, keepdims=True)
  a_d = jnp.sum(xl_h * adv_ref[pl.ds(h, 1), :], axis=1, keepdims=True)
  kh = jnp.maximum(jnp.max(a_s) + jnp.max(a_d), 0.0)
  e_self = a_s + a_d
  e_self = jnp.where(e_self >= 0, e_self, 0.2 * e_self)
  acc_col(asrc_ref, a_s, laneH)
  acc_col(adst_ref, a_d, laneH)
  acc_col(wself_ref, jnp.exp(e_self - kh), laneH)
  acc_col(k_ref, kh, lane16)


def _tc_b2(x1, w2, att_src, att_dst):
  full = lambda h: (0, 0)
  return pl.pallas_call(
      _tc_b2_body,
      grid=(H,),
      in_specs=[pl.BlockSpec((N, C), full),
                pl.BlockSpec((C, C), lambda h: (0, h)),
                pl.BlockSpec((H, C), full),
                pl.BlockSpec((H, C), full)],
      out_specs=[pl.BlockSpec((1, N, C), lambda h: (h, 0, 0)),
                 pl.BlockSpec((N, H), full),
                 pl.BlockSpec((N, H), full),
                 pl.BlockSpec((1, 16), full),
                 pl.BlockSpec((N, H), full)],
      out_shape=[
          jax.ShapeDtypeStruct((H, N, C), _f32),
          jax.ShapeDtypeStruct((N, H), _f32),
          jax.ShapeDtypeStruct((N, H), _f32),
          jax.ShapeDtypeStruct((1, 16), _f32),
          jax.ShapeDtypeStruct((N, H), _f32),
      ],
  )(x1, w2, att_src, att_dst)


def _tc_c_body(dp_ref, ws_ref, rden_ref, wsr_ref):
  den = (dp_ref[0, :N, 0:H] + dp_ref[1, :N, 0:H] + ws_ref[...] + 1e-16)
  rden = 0.25 / den
  rden_ref[...] = rden
  wsr_ref[...] = ws_ref[...] * rden


def _tc_c(denpart, wself):
  return pl.pallas_call(
      _tc_c_body,
      out_shape=[
          jax.ShapeDtypeStruct((N, H), _f32),
          jax.ShapeDtypeStruct((N, H), _f32),
      ],
  )(denpart, wself)


_DBLK = 1000


def _tc_d_body(a0_ref, a1_ref, a2_ref, a3_ref, xlh_ref, wsr_ref, dinv_ref,
               w3_ref, b2_ref, y3_ref):
  att = (a0_ref, a1_ref, a2_ref, a3_ref)
  acc = jnp.zeros((_DBLK, C), _f32)
  for h in range(H):
    a = att[h]
    acc = acc + a[0] + a[1] + wsr_ref[:, h:h + 1] * xlh_ref[h]
  x2 = jnp.maximum(acc + b2_ref[...], 0.0)
  y3_ref[...] = dinv_ref[...] * _dot(x2, w3_ref[...])


def _tc_d(attps, xlh, wsr, dinv, w3, b2):
  att_spec = pl.BlockSpec((2, _DBLK, C), lambda i: (0, i, 0))
  return pl.pallas_call(
      _tc_d_body,
      grid=(N // _DBLK,),
      in_specs=[att_spec, att_spec, att_spec, att_spec,
                pl.BlockSpec((H, _DBLK, C), lambda i: (0, i, 0)),
                pl.BlockSpec((_DBLK, H), lambda i: (i, 0)),
                pl.BlockSpec((_DBLK, 1), lambda i: (i, 0)),
                pl.BlockSpec((C, C), lambda i: (0, 0)),
                pl.BlockSpec((1, C), lambda i: (0, 0))],
      out_specs=pl.BlockSpec((_DBLK, C), lambda i: (i, 0)),
      out_shape=jax.ShapeDtypeStruct((N, C), _f32),
  )(*attps, xlh, wsr, dinv, w3, b2)


def _tc_e1_body(sp_ref, y3_ref, dinv_ref, b3_ref, x_ref, wsk_ref, bsk_ref,
                x3_ref):
  s3 = sp_ref[0, :N, :] + sp_ref[1, :N, :] + y3_ref[...]
  x3_ref[...] = (dinv_ref[...] * s3 + b3_ref[...]
                 + _dot(x_ref[...], wsk_ref[...]) + bsk_ref[...])


def _tc_e1(s3part, y3, dinv, b3, x, wsk, bsk):
  return pl.pallas_call(
      _tc_e1_body,
      out_shape=jax.ShapeDtypeStruct((N, C), _f32),
  )(s3part, y3, dinv, b3, x, wsk, bsk)


def _tc_e2_body(x3_ref, batch_ref, wf_ref, bf_ref, out_ref):
  gids = lax.broadcasted_iota(jnp.int32, (NUM_GRAPHS, N), 0).astype(_f32)
  mask = (batch_ref[...] == gids).astype(_f32)
  sums = _dot(mask, x3_ref[...])
  cnts = jnp.sum(mask, axis=1, keepdims=True)
  pooled = sums / jnp.maximum(cnts, 1.0)
  out_ref[...] = _dot(pooled, wf_ref[...]) + bf_ref[...]


def _tc_e2(x3, batch, wf, bf):
  return pl.pallas_call(
      _tc_e2_body,
      out_shape=jax.ShapeDtypeStruct((NUM_GRAPHS, NUM_CLASSES), _f32),
  )(x3, batch, wf, bf)


def kernel(x, edge_index, batch, W1, b1, W2, att_src, att_dst, b2, W3, b3,
           Wsk, bsk, Wf, bf):
  i32 = jnp.int32
  srcp = jnp.concatenate([edge_index[0], jnp.zeros((EPAD,), i32)])
  dstp = jnp.concatenate([edge_index[1], jnp.full((EPAD,), N, i32)])
  ones128 = jnp.ones((CHUNK, C), _f32)
  z128 = jnp.zeros((CHUNK, C), _f32)

  degpart = _sc_deg(dstp, ones128, z128)
  y1, dinv = _tc_a(x, W1, degpart)
  s1part = _sc_seg(y1, srcp, dstp, z128)
  x1 = _tc_b1(s1part, y1, dinv, b1.reshape(1, C))
  xlh, asrc, adst, k16, wself = _tc_b2(x1, W2, att_src, att_dst)
  k16f = k16.reshape(16)
  denflat, w4 = _sc_attden(asrc.reshape(N * H), adst.reshape(N * H), k16f,
                           srcp, dstp, z128)
  denpart = denflat.reshape(NC, NP, 16)
  rden4, wsr = _tc_c(denpart, wself)
  attps = []
  tok = jnp.zeros((), _f32)
  for h in range(H):
    ap = _sc_attagg(xlh[h], w4[h] + tok, rden4[:, h].reshape(N),
                    srcp, dstp, z128)
    attps.append(ap)
    tok = ap[0, 0, 0] * 0.0
  y3 = _tc_d(attps, xlh, wsr, dinv, W3, b2.reshape(1, C))
  s3part = _sc_seg(y3, srcp, dstp, z128)
  x3 = _tc_e1(s3part, y3, dinv, b3.reshape(1, C), x, Wsk, bsk.reshape(1, C))
  sums = jax.ops.segment_sum(x3, batch, num_segments=NUM_GRAPHS)
  cnts = jax.ops.segment_sum(jnp.ones((N,), _f32), batch,
                             num_segments=NUM_GRAPHS)
  pooled = sums / jnp.maximum(cnts, 1.0)[:, None]
  return pooled @ Wf + bf


# speedup vs baseline: 14.7987x; 1.0163x over previous
import dataclasses
import functools

import jax
import jax.numpy as jnp
from jax import lax
from jax.experimental import pallas as pl
from jax.experimental.pallas import tpu as pltpu
from jax.experimental.pallas import tpu_sc as plsc

N = 10000
E = 320000
D = 128
H = 4
C = 128
NUM_CLASSES = 64
NUM_GRAPHS = 16

NC = 2
NS = 16
NW = NC * NS
CHUNK = 128
NCHUNK = 79
EPW = CHUNK * NCHUNK
EPAD = NW * EPW - E
NP = 10240
RPT = NP // NS

_f32 = jnp.float32
_mesh = plsc.VectorSubcoreMesh(
    core_axis_name="c", subcore_axis_name="s", num_cores=NC, num_subcores=NS)

_sc_params = pltpu.CompilerParams()
if "needs_layout_passes" in pltpu.CompilerParams.__dataclass_fields__:
  _sc_params = dataclasses.replace(_sc_params, needs_layout_passes=False)


def _worker_id():
  return lax.axis_index("s") * NC + lax.axis_index("c")


def _splat(vref, i):
  return plsc.load_gather(vref, [jnp.full((16,), i, jnp.int32)])


@functools.partial(
    pl.kernel,
    out_type=jax.ShapeDtypeStruct((NC, NP, C), _f32),
    mesh=_mesh,
    compiler_params=_sc_params,
    scratch_types=[
        pltpu.VMEM((CHUNK,), jnp.int32),
        pltpu.VMEM((CHUNK, C), _f32),
        pltpu.VMEM_SHARED((NP, C), _f32),
    ],
)
def _sc_deg(dst_hbm, ones_hbm, z128_hbm, out_hbm, didx, ones_v, acc):
  c = lax.axis_index("c")
  s = lax.axis_index("s")
  wid = _worker_id()
  pltpu.sync_copy(z128_hbm, ones_v)

  @pl.loop(0, 5)
  def _zero(i):
    pltpu.sync_copy(ones_v, acc.at[pl.ds(s * RPT + i * CHUNK, CHUNK)])

  pltpu.sync_copy(ones_hbm, ones_v)
  plsc.subcore_barrier()

  @pl.loop(0, NCHUNK)
  def _edges(k):
    pltpu.sync_copy(dst_hbm.at[pl.ds(wid * EPW + k * CHUNK, CHUNK)], didx)
    pltpu.sync_copy(ones_v, acc.at[didx], add=True)

  plsc.subcore_barrier()

  @pl.loop(0, 5)
  def _wb(i):
    pltpu.sync_copy(acc.at[pl.ds(s * RPT + i * CHUNK, CHUNK)], ones_v)
    pltpu.sync_copy(ones_v, out_hbm.at[c, pl.ds(s * RPT + i * CHUNK, CHUNK)])


@functools.partial(
    pl.kernel,
    out_type=jax.ShapeDtypeStruct((NC, NP, C), _f32),
    mesh=_mesh,
    compiler_params=_sc_params,
    scratch_types=[
        pltpu.VMEM((CHUNK,), jnp.int32),
        pltpu.VMEM((CHUNK,), jnp.int32),
        pltpu.VMEM((CHUNK, C), _f32),
        pltpu.VMEM_SHARED((NP, C), _f32),
    ],
)
def _sc_seg(y_hbm, src_hbm, dst_hbm, z128_hbm, out_hbm, sidx, didx, rows, acc):
  c = lax.axis_index("c")
  s = lax.axis_index("s")
  wid = _worker_id()
  pltpu.sync_copy(z128_hbm, rows)

  @pl.loop(0, 5)
  def _zero(i):
    pltpu.sync_copy(rows, acc.at[pl.ds(s * RPT + i * CHUNK, CHUNK)])

  plsc.subcore_barrier()

  @pl.loop(0, NCHUNK)
  def _edges(k):
    base = wid * EPW + k * CHUNK
    pltpu.sync_copy(src_hbm.at[pl.ds(base, CHUNK)], sidx)
    pltpu.sync_copy(dst_hbm.at[pl.ds(base, CHUNK)], didx)
    pltpu.sync_copy(y_hbm.at[sidx], rows)
    pltpu.sync_copy(rows, acc.at[didx], add=True)

  plsc.subcore_barrier()

  @pl.loop(0, 5)
  def _wb(i):
    pltpu.sync_copy(acc.at[pl.ds(s * RPT + i * CHUNK, CHUNK)], rows)
    pltpu.sync_copy(rows, out_hbm.at[c, pl.ds(s * RPT + i * CHUNK, CHUNK)])


NPK = NP // 8
PKT = NPK // NS
UPT = NP // NS * 16


@functools.partial(
    pl.kernel,
    out_type=[
        jax.ShapeDtypeStruct((NC, NS * UPT), _f32),
        jax.ShapeDtypeStruct((H, NW * EPW), _f32),
    ],
    mesh=_mesh,
    compiler_params=_sc_params,
    scratch_types=[
        pltpu.VMEM((CHUNK,), jnp.int32),
        pltpu.VMEM((CHUNK,), jnp.int32),
        pltpu.VMEM((CHUNK,), jnp.int32),
        pltpu.VMEM((CHUNK, C), _f32),
        pltpu.VMEM((H * CHUNK,), _f32),
        pltpu.VMEM((N * H,), _f32),
        pltpu.VMEM((N * H,), _f32),
        pltpu.VMEM((16,), _f32),
        pltpu.VMEM((PKT, C), _f32),
        pltpu.VMEM((UPT,), _f32),
        pltpu.VMEM_SHARED((NPK, C), _f32),
    ],
)
def _sc_attden(asrc_hbm, adst_hbm, k16_hbm, src_hbm, dst_hbm, z128_hbm,
               out_hbm, w_hbm, sidx, didx, didx8, wrow, wout, at_s, at_d, kv,
               pk, ub, acc):
  c = lax.axis_index("c")
  s = lax.axis_index("s")
  wid = _worker_id()
  pltpu.sync_copy(asrc_hbm, at_s)
  pltpu.sync_copy(adst_hbm, at_d)
  pltpu.sync_copy(k16_hbm, kv)
  pltpu.sync_copy(z128_hbm.at[pl.ds(0, PKT)], pk)
  pltpu.sync_copy(pk, acc.at[pl.ds(s * PKT, PKT)])

  @pl.loop(0, CHUNK)
  def _zrow(i):
    for r in range(C // 16):
      wrow[i, pl.ds(r * 16, 16)] = jnp.zeros((16,), _f32)

  plsc.subcore_barrier()
  lanes = lax.iota(jnp.int32, 16)
  zeros16 = jnp.zeros((16,), _f32)

  @pl.loop(0, NCHUNK)
  def _edges(k):
    base = wid * EPW + k * CHUNK
    pltpu.sync_copy(src_hbm.at[pl.ds(base, CHUNK)], sidx)
    pltpu.sync_copy(dst_hbm.at[pl.ds(base, CHUNK)], didx)

    @pl.loop(0, CHUNK // 16)
    def _group(g):
      s16 = sidx[pl.ds(g * 16, 16)]
      d16 = didx[pl.ds(g * 16, 16)]
      didx8[pl.ds(g * 16, 16)] = lax.shift_right_logical(d16, 3)
      rowids = g * 16 + lanes
      col = lax.shift_left(jnp.bitwise_and(d16, 7), 4)
      for h in range(H):
        e = (plsc.load_gather(at_s, [s16 * H + h])
             + plsc.load_gather(at_d, [d16 * H + h]))
        e = jnp.where(e >= 0, e, 0.2 * e)
        w = jnp.exp(e - _splat(kv, h))
        wout[pl.ds(h * CHUNK + g * 16, 16)] = w
        plsc.store_scatter(wrow, [rowids, col + h], w)

    pltpu.sync_copy(wrow, acc.at[didx8], add=True)
    for h in range(H):
      pltpu.sync_copy(wout.at[pl.ds(h * CHUNK, CHUNK)],
                      w_hbm.at[h, pl.ds(base, CHUNK)])

    @pl.loop(0, CHUNK // 16)
    def _rezero(g):
      d16 = didx[pl.ds(g * 16, 16)]
      rowids = g * 16 + lanes
      col = lax.shift_left(jnp.bitwise_and(d16, 7), 4)
      for h in range(H):
        plsc.store_scatter(wrow, [rowids, col + h], zeros16)

  plsc.subcore_barrier()
  pltpu.sync_copy(acc.at[pl.ds(s * PKT, PKT)], pk)

  @pl.loop(0, NP // NS)
  def _unpack(i):
    r = lax.shift_right_logical(i, 3)
    q = lax.shift_left(jnp.bitwise_and(i, 7), 4)
    ub[pl.ds(i * 16, 16)] = pk[r, pl.ds(q, 16)]

  pltpu.sync_copy(ub, out_hbm.at[c, pl.ds(s * UPT, UPT)])


@functools.partial(
    pl.kernel,
    out_type=jax.ShapeDtypeStruct((NC, NP, C), _f32),
    mesh=_mesh,
    compiler_params=_sc_params,
    scratch_types=[
        pltpu.VMEM((CHUNK,), jnp.int32),
        pltpu.VMEM((CHUNK,), jnp.int32),
        pltpu.VMEM((CHUNK, C), _f32),
        pltpu.VMEM((CHUNK,), _f32),
        pltpu.VMEM((N,), _f32),
        pltpu.VMEM((16,), _f32),
        pltpu.VMEM_SHARED((NP, C), _f32),
    ],
)
def _sc_attagg(xlh_hbm, wh_hbm, rden_hbm, src_hbm, dst_hbm, z128_hbm,
               out_hbm, sidx, didx, rows, wchunk, rden, vbuf, acc):
  c = lax.axis_index("c")
  s = lax.axis_index("s")
  wid = _worker_id()
  pltpu.sync_copy(rden_hbm, rden)
  pltpu.sync_copy(z128_hbm, rows)

  @pl.loop(0, 5)
  def _zero(i):
    pltpu.sync_copy(rows, acc.at[pl.ds(s * RPT + i * CHUNK, CHUNK)])

  plsc.subcore_barrier()

  @pl.loop(0, NCHUNK)
  def _edges(k):
    base = wid * EPW + k * CHUNK
    pltpu.sync_copy(src_hbm.at[pl.ds(base, CHUNK)], sidx)
    pltpu.sync_copy(dst_hbm.at[pl.ds(base, CHUNK)], didx)
    pltpu.sync_copy(wh_hbm.at[pl.ds(base, CHUNK)], wchunk)
    pltpu.sync_copy(xlh_hbm.at[sidx], rows)

    @pl.loop(0, CHUNK // 16)
    def _group(g):
      d16 = didx[pl.ds(g * 16, 16)]
      w16 = wchunk[pl.ds(g * 16, 16)]
      vbuf[...] = w16 * plsc.load_gather(rden, [d16])
      for j in range(16):
        vj = _splat(vbuf, j)
        eidx = g * 16 + j
        for r in range(C // 16):
          rows[eidx, pl.ds(r * 16, 16)] = vj * rows[eidx, pl.ds(r * 16, 16)]

    pltpu.sync_copy(rows, acc.at[didx], add=True)

  plsc.subcore_barrier()

  @pl.loop(0, 5)
  def _wb(i):
    pltpu.sync_copy(acc.at[pl.ds(s * RPT + i * CHUNK, CHUNK)], rows)
    pltpu.sync_copy(rows, out_hbm.at[c, pl.ds(s * RPT + i * CHUNK, CHUNK)])


_HI = jax.lax.Precision.HIGHEST


def _dot(a, b):
  return jax.lax.dot_general(a, b, (((1,), (0,)), ((), ())), precision=_HI,
                             preferred_element_type=_f32)


def _tc_a_body(x_ref, w1_ref, dp_ref, y1_ref, dinv_ref):
  deg = dp_ref[0, :N, 0:1] + dp_ref[1, :N, 0:1] + 1.0
  dinv = jax.lax.rsqrt(deg)
  dinv_ref[...] = dinv
  y1_ref[...] = dinv * _dot(x_ref[...], w1_ref[...])


def _tc_a(x, w1, degpart):
  return pl.pallas_call(
      _tc_a_body,
      out_shape=[
          jax.ShapeDtypeStruct((N, C), _f32),
          jax.ShapeDtypeStruct((N, 1), _f32),
      ],
  )(x, w1, degpart)


def _tc_b1_body(sp_ref, y1_ref, dinv_ref, b1_ref, x1_ref):
  s1 = sp_ref[0, :N, :] + sp_ref[1, :N, :] + y1_ref[...]
  x1_ref[...] = jnp.maximum(dinv_ref[...] * s1 + b1_ref[...], 0.0)


def _tc_b1(s1part, y1, dinv, b1):
  return pl.pallas_call(
      _tc_b1_body,
      out_shape=jax.ShapeDtypeStruct((N, C), _f32),
  )(s1part, y1, dinv, b1)


def _tc_b2_body(x1_ref, w2_ref, asv_ref, adv_ref,
                xlh_ref, asrc_ref, adst_ref, k_ref, wself_ref):
  h = pl.program_id(0)
  laneH = lax.broadcasted_iota(jnp.int32, (N, H), 1)
  lane16 = lax.broadcasted_iota(jnp.int32, (1, 16), 1)

  def acc_col(ref, val, lane):
    cur = jnp.where(lane == h, val, 0.0)

    @pl.when(h == 0)
    def _init():
      ref[...] = cur

    @pl.when(h != 0)
    def _add():
      ref[...] = ref[...] + cur

  xl_h = _dot(x1_ref[...], w2_ref[...])
  xlh_ref[0] = xl_h
  a_s = jnp.sum(xl_h * asv_ref[pl.ds(h, 1), :], axis=1, keepdims=True)
  a_d = jnp.sum(xl_h * adv_ref[pl.ds(h, 1), :], axis=1, keepdims=True)
  kh = jnp.maximum(jnp.max(a_s) + jnp.max(a_d), 0.0)
  e_self = a_s + a_d
  e_self = jnp.where(e_self >= 0, e_self, 0.2 * e_self)
  acc_col(asrc_ref, a_s, laneH)
  acc_col(adst_ref, a_d, laneH)
  acc_col(wself_ref, jnp.exp(e_self - kh), laneH)
  acc_col(k_ref, kh, lane16)


def _tc_b2(x1, w2, att_src, att_dst):
  full = lambda h: (0, 0)
  return pl.pallas_call(
      _tc_b2_body,
      grid=(H,),
      in_specs=[pl.BlockSpec((N, C), full),
                pl.BlockSpec((C, C), lambda h: (0, h)),
                pl.BlockSpec((H, C), full),
                pl.BlockSpec((H, C), full)],
      out_specs=[pl.BlockSpec((1, N, C), lambda h: (h, 0, 0)),
                 pl.BlockSpec((N, H), full),
                 pl.BlockSpec((N, H), full),
                 pl.BlockSpec((1, 16), full),
                 pl.BlockSpec((N, H), full)],
      out_shape=[
          jax.ShapeDtypeStruct((H, N, C), _f32),
          jax.ShapeDtypeStruct((N, H), _f32),
          jax.ShapeDtypeStruct((N, H), _f32),
          jax.ShapeDtypeStruct((1, 16), _f32),
          jax.ShapeDtypeStruct((N, H), _f32),
      ],
  )(x1, w2, att_src, att_dst)


def _tc_c_body(dp_ref, ws_ref, rden_ref, wsr_ref):
  den = (dp_ref[0, :N, 0:H] + dp_ref[1, :N, 0:H] + ws_ref[...] + 1e-16)
  rden = 0.25 / den
  rden_ref[...] = rden
  wsr_ref[...] = ws_ref[...] * rden


def _tc_c(denpart, wself):
  return pl.pallas_call(
      _tc_c_body,
      out_shape=[
          jax.ShapeDtypeStruct((N, H), _f32),
          jax.ShapeDtypeStruct((N, H), _f32),
      ],
  )(denpart, wself)


_DBLK = 1000


def _tc_d_body(a0_ref, a1_ref, a2_ref, a3_ref, xlh_ref, wsr_ref, dinv_ref,
               w3_ref, b2_ref, y3_ref):
  att = (a0_ref, a1_ref, a2_ref, a3_ref)
  acc = jnp.zeros((_DBLK, C), _f32)
  for h in range(H):
    a = att[h]
    acc = acc + a[0] + a[1] + wsr_ref[:, h:h + 1] * xlh_ref[h]
  x2 = jnp.maximum(acc + b2_ref[...], 0.0)
  y3_ref[...] = dinv_ref[...] * _dot(x2, w3_ref[...])


def _tc_d(attps, xlh, wsr, dinv, w3, b2):
  att_spec = pl.BlockSpec((2, _DBLK, C), lambda i: (0, i, 0))
  return pl.pallas_call(
      _tc_d_body,
      grid=(N // _DBLK,),
      in_specs=[att_spec, att_spec, att_spec, att_spec,
                pl.BlockSpec((H, _DBLK, C), lambda i: (0, i, 0)),
                pl.BlockSpec((_DBLK, H), lambda i: (i, 0)),
                pl.BlockSpec((_DBLK, 1), lambda i: (i, 0)),
                pl.BlockSpec((C, C), lambda i: (0, 0)),
                pl.BlockSpec((1, C), lambda i: (0, 0))],
      out_specs=pl.BlockSpec((_DBLK, C), lambda i: (i, 0)),
      out_shape=jax.ShapeDtypeStruct((N, C), _f32),
  )(*attps, xlh, wsr, dinv, w3, b2)


def _tc_e1_body(sp_ref, y3_ref, dinv_ref, b3_ref, x_ref, wsk_ref, bsk_ref,
                x3_ref):
  s3 = sp_ref[0, :N, :] + sp_ref[1, :N, :] + y3_ref[...]
  x3_ref[...] = (dinv_ref[...] * s3 + b3_ref[...]
                 + _dot(x_ref[...], wsk_ref[...]) + bsk_ref[...])


def _tc_e1(s3part, y3, dinv, b3, x, wsk, bsk):
  return pl.pallas_call(
      _tc_e1_body,
      out_shape=jax.ShapeDtypeStruct((N, C), _f32),
  )(s3part, y3, dinv, b3, x, wsk, bsk)


def _tc_e2_body(x3_ref, batch_ref, wf_ref, bf_ref, out_ref):
  gids = lax.broadcasted_iota(jnp.int32, (NUM_GRAPHS, N), 0).astype(_f32)
  mask = (batch_ref[...] == gids).astype(_f32)
  sums = _dot(mask, x3_ref[...])
  cnts = jnp.sum(mask, axis=1, keepdims=True)
  pooled = sums / jnp.maximum(cnts, 1.0)
  out_ref[...] = _dot(pooled, wf_ref[...]) + bf_ref[...]


def _tc_e2(x3, batch, wf, bf):
  return pl.pallas_call(
      _tc_e2_body,
      out_shape=jax.ShapeDtypeStruct((NUM_GRAPHS, NUM_CLASSES), _f32),
  )(x3, batch, wf, bf)


def kernel(x, edge_index, batch, W1, b1, W2, att_src, att_dst, b2, W3, b3,
           Wsk, bsk, Wf, bf):
  i32 = jnp.int32
  srcp = jnp.concatenate([edge_index[0], jnp.zeros((EPAD,), i32)])
  dstp = jnp.concatenate([edge_index[1], jnp.full((EPAD,), N, i32)])
  ones128 = jnp.ones((CHUNK, C), _f32)
  z128 = jnp.zeros((CHUNK, C), _f32)

  degpart = _sc_deg(dstp, ones128, z128)
  y1, dinv = _tc_a(x, W1, degpart)
  s1part = _sc_seg(y1, srcp, dstp, z128)
  x1 = _tc_b1(s1part, y1, dinv, b1.reshape(1, C))
  xlh, asrc, adst, k16, wself = _tc_b2(x1, W2, att_src, att_dst)
  k16f = k16.reshape(16)
  denflat, w4 = _sc_attden(asrc.reshape(N * H), adst.reshape(N * H), k16f,
                           srcp, dstp, z128)
  denpart = denflat.reshape(NC, NP, 16)
  rden4, wsr = _tc_c(denpart, wself)
  attps = []
  tok = jnp.zeros((), _f32)
  for h in range(H):
    ap = _sc_attagg(xlh[h], w4[h] + tok, rden4[:, h].reshape(N),
                    srcp, dstp, z128)
    attps.append(ap)
    tok = ap[0, 0, 0] * 0.0
  y3 = _tc_d(attps, xlh, wsr, dinv, W3, b2.reshape(1, C))
  s3part = _sc_seg(y3, srcp, dstp, z128)
  x3 = _tc_e1(s3part, y3, dinv, b3.reshape(1, C), x, Wsk, bsk.reshape(1, C))
  sums = jax.ops.segment_sum(x3, batch, num_segments=NUM_GRAPHS)
  cnts = jax.ops.segment_sum(jnp.ones((N,), _f32), batch,
                             num_segments=NUM_GRAPHS)
  pooled = sums / jnp.maximum(cnts, 1.0)[:, None]
  return pooled @ Wf + bf
